# Initial kernel scaffold; baseline (speedup 1.0000x reference)
import jax
import jax.numpy as jnp
from jax.experimental import pallas as pl


def kernel(x, edge_index, edge_weight, W1, b1, W2, b2):
    raise NotImplementedError("write your pallas kernel here")


# speedup vs baseline: 14.1025x; 14.1025x over previous
import dataclasses
import functools

import jax
import jax.numpy as jnp
from jax import lax
from jax.experimental import pallas as pl
from jax.experimental.pallas import tpu as pltpu
from jax.experimental.pallas import tpu_sc as plsc

N = 10000
E = 320000
D = 128
NP = 10240
NC, NS = 2, 16
NW = NC * NS
EPW = E // NW
CH = 80
NCHUNK = EPW // CH
BCH = 25
NBLK = NCHUNK // BCH
EPT = E // NS
DEGC = EPT // 16
SLICE = NP // NS
RPT = N // NS
ZROWS = 25
BM = 1000


def _sc_mesh():
    return plsc.VectorSubcoreMesh(
        core_axis_name="c", subcore_axis_name="s", num_cores=NC, num_subcores=NS
    )


def _sc_params():
    cp = pltpu.CompilerParams()
    fields = pltpu.CompilerParams.__dataclass_fields__
    if "needs_layout_passes" in fields:
        cp = dataclasses.replace(cp, needs_layout_passes=False)
    if "use_tc_tiling_on_sc" in fields:
        cp = dataclasses.replace(cp, use_tc_tiling_on_sc=False)
    return cp


def _rsqrt16(x):
    i = plsc.bitcast(x, jnp.int32)
    y = plsc.bitcast(jnp.int32(0x5F3759DF) - (i >> 1), jnp.float32)
    for _ in range(4):
        y = y * (1.5 - 0.5 * x * y * y)
    return y


def _deg_g(cols2, ew2):
    @functools.partial(
        pl.kernel,
        out_type=jax.ShapeDtypeStruct((NP,), jnp.float32),
        mesh=_sc_mesh(),
        scratch_types=[
            pltpu.VMEM((DEGC, 16), jnp.int32),
            pltpu.VMEM((DEGC, 16), jnp.float32),
            pltpu.VMEM((NP,), jnp.float32),
            pltpu.VMEM((NS, SLICE), jnp.float32),
            pltpu.VMEM((SLICE,), jnp.float32),
            pltpu.VMEM_SHARED((NS, NP), jnp.float32),
            pltpu.SemaphoreType.DMA,
        ],
        compiler_params=_sc_params(),
    )
    def deg_k(cols_hbm, ew_hbm, g_hbm, colv, ewv, degl, redv, gv, degsh, sem):
        cid = lax.axis_index("c")
        sid = lax.axis_index("s")

        @pl.when(cid == 0)
        def _():
            pltpu.async_copy(cols_hbm.at[sid], colv, sem).wait()
            pltpu.async_copy(ew_hbm.at[sid], ewv, sem).wait()

            @pl.loop(0, NP // 16)
            def _(i):
                degl[pl.ds(i * 16, 16)] = jnp.zeros((16,), jnp.float32)

            @pl.loop(0, DEGC)
            def _(j):
                plsc.addupdate_scatter(degl, [colv[j]], ewv[j])

            pltpu.sync_copy(degl, degsh.at[sid])
            plsc.subcore_barrier()

            pltpu.sync_copy(degsh.at[:, pl.ds(sid * SLICE, SLICE)], redv)

            @pl.loop(0, SLICE // 16)
            def _(m):
                acc = jnp.full((16,), 1.0, jnp.float32)
                for p in range(NS):
                    acc = acc + redv[p, pl.ds(m * 16, 16)]
                gv[pl.ds(m * 16, 16)] = _rsqrt16(acc)

            pltpu.sync_copy(gv, g_hbm.at[pl.ds(sid * SLICE, SLICE)])

    return deg_k(cols2, ew2)


def _agg(table, rows3, cols3, ew3, g):
    @functools.partial(
        pl.kernel,
        out_type=jax.ShapeDtypeStruct((NC, N, D), jnp.float32),
        mesh=_sc_mesh(),
        scratch_types=[
            pltpu.VMEM((BCH, CH), jnp.int32),
            pltpu.VMEM((BCH, CH), jnp.int32),
            pltpu.VMEM((BCH, CH), jnp.float32),
            pltpu.VMEM((NP,), jnp.float32),
            pltpu.VMEM((CH,), jnp.float32),
            pltpu.VMEM((CH, D), jnp.float32),
            pltpu.VMEM((ZROWS, D), jnp.float32),
            pltpu.VMEM_SHARED((N, D), jnp.float32),
            pltpu.SemaphoreType.DMA,
        ],
        compiler_params=_sc_params(),
    )
    def agg_k(t_hbm, r_hbm, c_hbm, w_hbm, g_hbm, out_hbm,
              rv, cv, wv, gv, sv, buf, zbuf, accum, sem):
        cid = lax.axis_index("c")
        sid = lax.axis_index("s")
        wid = cid * NS + sid

        pltpu.async_copy(g_hbm, gv, sem).wait()

        @pl.loop(0, ZROWS)
        def _(i):
            for k in range(D // 16):
                zbuf[i, pl.ds(k * 16, 16)] = jnp.zeros((16,), jnp.float32)

        @pl.loop(0, RPT // ZROWS)
        def _(m):
            pltpu.sync_copy(zbuf, accum.at[pl.ds(sid * RPT + m * ZROWS, ZROWS)])
        plsc.subcore_barrier()

        @pl.loop(0, NBLK)
        def _(b):
            pltpu.async_copy(r_hbm.at[wid, b], rv, sem).wait()
            pltpu.async_copy(c_hbm.at[wid, b], cv, sem).wait()
            pltpu.async_copy(w_hbm.at[wid, b], wv, sem).wait()

            @pl.loop(0, BCH)
            def _(j):
                gather = pltpu.async_copy(t_hbm.at[rv.at[j]], buf, sem)
                for t in range(CH // 16):
                    r16 = rv[j, pl.ds(t * 16, 16)]
                    s16 = wv[j, pl.ds(t * 16, 16)] * plsc.load_gather(gv, [r16])
                    sv[pl.ds(t * 16, 16)] = s16
                gather.wait()

                @pl.loop(0, CH)
                def _(i):
                    sc = plsc.load_gather(sv, [jnp.broadcast_to(i, (16,))])
                    for k in range(D // 16):
                        sl = pl.ds(k * 16, 16)
                        buf[i, sl] = buf[i, sl] * sc

                pltpu.sync_copy(buf, accum.at[cv.at[j]], add=True)

        plsc.subcore_barrier()
        pltpu.sync_copy(
            accum.at[pl.ds(sid * RPT, RPT)],
            out_hbm.at[cid, pl.ds(sid * RPT, RPT)],
        )

    return agg_k(table, rows3, cols3, ew3, g)


def _dot(a, b):
    return lax.dot_general(
        a, b, (((1,), (0,)), ((), ())),
        precision=lax.Precision.HIGHEST,
        preferred_element_type=jnp.float32,
    )


def _mm(x, W):
    def body(x_ref, w_ref, o_ref):
        o_ref[...] = _dot(x_ref[...], w_ref[...])

    return pl.pallas_call(
        body,
        grid=(N // BM,),
        in_specs=[
            pl.BlockSpec((BM, D), lambda i: (i, 0)),
            pl.BlockSpec((D, D), lambda i: (0, 0)),
        ],
        out_specs=pl.BlockSpec((BM, D), lambda i: (i, 0)),
        out_shape=jax.ShapeDtypeStruct((N, D), jnp.float32),
    )(x, W)


def _mid(A, H1, g2, b1, W2):
    def body(a_ref, h_ref, g_ref, b_ref, w_ref, o_ref):
        g = g_ref[...]
        o = g * (a_ref[0] + a_ref[1] + g * h_ref[...]) + b_ref[...]
        o = jnp.where(o >= 0, o, 0.01 * o)
        o_ref[...] = _dot(o, w_ref[...])

    return pl.pallas_call(
        body,
        grid=(N // BM,),
        in_specs=[
            pl.BlockSpec((NC, BM, D), lambda i: (0, i, 0)),
            pl.BlockSpec((BM, D), lambda i: (i, 0)),
            pl.BlockSpec((BM, 1), lambda i: (i, 0)),
            pl.BlockSpec((1, D), lambda i: (0, 0)),
            pl.BlockSpec((D, D), lambda i: (0, 0)),
        ],
        out_specs=pl.BlockSpec((BM, D), lambda i: (i, 0)),
        out_shape=jax.ShapeDtypeStruct((N, D), jnp.float32),
    )(A, H1, g2, b1, W2)


def _final(A, H2, g2, b2):
    def body(a_ref, h_ref, g_ref, b_ref, o_ref):
        g = g_ref[...]
        o_ref[...] = g * (a_ref[0] + a_ref[1] + g * h_ref[...]) + b_ref[...]

    return pl.pallas_call(
        body,
        grid=(N // BM,),
        in_specs=[
            pl.BlockSpec((NC, BM, D), lambda i: (0, i, 0)),
            pl.BlockSpec((BM, D), lambda i: (i, 0)),
            pl.BlockSpec((BM, 1), lambda i: (i, 0)),
            pl.BlockSpec((1, D), lambda i: (0, 0)),
        ],
        out_specs=pl.BlockSpec((BM, D), lambda i: (i, 0)),
        out_shape=jax.ShapeDtypeStruct((N, D), jnp.float32),
    )(A, H2, g2, b2)


def kernel(x, edge_index, edge_weight, W1, b1, W2, b2):
    rows = edge_index[0].astype(jnp.int32)
    cols = edge_index[1].astype(jnp.int32)
    ew = edge_weight.astype(jnp.float32)

    rows3 = rows.reshape(NW, NBLK, BCH, CH)
    cols3 = cols.reshape(NW, NBLK, BCH, CH)
    ew3 = ew.reshape(NW, NBLK, BCH, CH)
    cols2 = cols.reshape(NS, DEGC, 16)
    ew2 = ew.reshape(NS, DEGC, 16)

    g_full = _deg_g(cols2, ew2)
    H1 = _mm(x, W1)
    A1 = _agg(H1, rows3, cols3, ew3, g_full)

    g2 = g_full[:N].reshape(N, 1)
    H2 = _mid(A1, H1, g2, b1.reshape(1, D), W2)
    A2 = _agg(H2, rows3, cols3, ew3, g_full)
    return _final(A2, H2, g2, b2.reshape(1, D))


# speedup vs baseline: 23.7044x; 1.6809x over previous
import dataclasses
import functools

import jax
import jax.numpy as jnp
from jax import lax
from jax.experimental import pallas as pl
from jax.experimental.pallas import tpu as pltpu
from jax.experimental.pallas import tpu_sc as plsc

N = 10000
E = 320000
D = 128
NP = 10240
NC, NS = 2, 16
NW = NC * NS
EPW = E // NW
CH = 80
NCHUNK = EPW // CH
BCH = 25
NBLK = NCHUNK // BCH
EPT = E // NS
DEGC = EPT // 16
SLICE = NP // NS
RPT = N // NS
ZROWS = 25
BM = 1000


def _sc_mesh():
    return plsc.VectorSubcoreMesh(
        core_axis_name="c", subcore_axis_name="s", num_cores=NC, num_subcores=NS
    )


def _sc_params():
    cp = pltpu.CompilerParams()
    fields = pltpu.CompilerParams.__dataclass_fields__
    if "needs_layout_passes" in fields:
        cp = dataclasses.replace(cp, needs_layout_passes=False)
    if "use_tc_tiling_on_sc" in fields:
        cp = dataclasses.replace(cp, use_tc_tiling_on_sc=False)
    return cp


def _rsqrt16(x):
    i = plsc.bitcast(x, jnp.int32)
    y = plsc.bitcast(jnp.int32(0x5F3759DF) - (i >> 1), jnp.float32)
    for _ in range(4):
        y = y * (1.5 - 0.5 * x * y * y)
    return y


def _deg_g(cols2, ew2):
    @functools.partial(
        pl.kernel,
        out_type=jax.ShapeDtypeStruct((NP,), jnp.float32),
        mesh=_sc_mesh(),
        scratch_types=[
            pltpu.VMEM((DEGC, 16), jnp.int32),
            pltpu.VMEM((DEGC, 16), jnp.float32),
            pltpu.VMEM((NP,), jnp.float32),
            pltpu.VMEM((NS, SLICE), jnp.float32),
            pltpu.VMEM((SLICE,), jnp.float32),
            pltpu.VMEM_SHARED((NS, NP), jnp.float32),
            pltpu.SemaphoreType.DMA,
        ],
        compiler_params=_sc_params(),
    )
    def deg_k(cols_hbm, ew_hbm, g_hbm, colv, ewv, degl, redv, gv, degsh, sem):
        cid = lax.axis_index("c")
        sid = lax.axis_index("s")

        @pl.when(cid == 0)
        def _():
            pltpu.async_copy(cols_hbm.at[sid], colv, sem).wait()
            pltpu.async_copy(ew_hbm.at[sid], ewv, sem).wait()

            @pl.loop(0, NP // 16)
            def _(i):
                degl[pl.ds(i * 16, 16)] = jnp.zeros((16,), jnp.float32)

            @pl.loop(0, DEGC)
            def _(j):
                plsc.addupdate_scatter(degl, [colv[j]], ewv[j])

            pltpu.sync_copy(degl, degsh.at[sid])
            plsc.subcore_barrier()

            pltpu.sync_copy(degsh.at[:, pl.ds(sid * SLICE, SLICE)], redv)

            @pl.loop(0, SLICE // 16)
            def _(m):
                acc = jnp.full((16,), 1.0, jnp.float32)
                for p in range(NS):
                    acc = acc + redv[p, pl.ds(m * 16, 16)]
                gv[pl.ds(m * 16, 16)] = _rsqrt16(acc)

            pltpu.sync_copy(gv, g_hbm.at[pl.ds(sid * SLICE, SLICE)])

    return deg_k(cols2, ew2)


def _agg(table, rows3, cols3, ew3, g):
    @functools.partial(
        pl.kernel,
        out_type=jax.ShapeDtypeStruct((NC, N, D), jnp.float32),
        mesh=_sc_mesh(),
        scratch_types=[
            pltpu.VMEM((2 * BCH, CH), jnp.int32),
            pltpu.VMEM((2 * BCH, CH), jnp.int32),
            pltpu.VMEM((2 * BCH, CH), jnp.float32),
            pltpu.VMEM((NP,), jnp.float32),
            pltpu.VMEM((CH,), jnp.float32),
            pltpu.VMEM((CH, D), jnp.float32),
            pltpu.VMEM((CH, D), jnp.float32),
            pltpu.VMEM((ZROWS, D), jnp.float32),
            pltpu.VMEM_SHARED((N, D), jnp.float32),
            pltpu.SemaphoreType.DMA,
            pltpu.SemaphoreType.DMA,
            pltpu.SemaphoreType.DMA,
            pltpu.SemaphoreType.DMA,
            pltpu.SemaphoreType.DMA,
        ],
        compiler_params=_sc_params(),
    )
    def agg_k(t_hbm, r_hbm, c_hbm, w_hbm, g_hbm, out_hbm,
              rv, cv, wv, gv, sv, buf0, buf1, zbuf, accum,
              bsem, gsem0, gsem1, ssem0, ssem1):
        cid = lax.axis_index("c")
        sid = lax.axis_index("s")
        wid = cid * NS + sid
        bufs = (buf0, buf1)
        gsems = (gsem0, gsem1)
        ssems = (ssem0, ssem1)

        pltpu.async_copy(g_hbm, gv, bsem).wait()

        @pl.loop(0, ZROWS)
        def _(i):
            for k in range(D // 16):
                zbuf[i, pl.ds(k * 16, 16)] = jnp.zeros((16,), jnp.float32)

        @pl.loop(0, RPT // ZROWS)
        def _(m):
            pltpu.sync_copy(zbuf, accum.at[pl.ds(sid * RPT + m * ZROWS, ZROWS)])
        plsc.subcore_barrier()

        def load_block(b, off):
            pltpu.async_copy(r_hbm.at[wid, b], rv.at[pl.ds(off, BCH)], bsem)
            pltpu.async_copy(c_hbm.at[wid, b], cv.at[pl.ds(off, BCH)], bsem)
            pltpu.async_copy(w_hbm.at[wid, b], wv.at[pl.ds(off, BCH)], bsem)

        def wait_block():
            pltpu.make_async_copy(r_hbm.at[wid, 0], rv.at[pl.ds(0, BCH)], bsem).wait()
            pltpu.make_async_copy(c_hbm.at[wid, 0], cv.at[pl.ds(0, BCH)], bsem).wait()
            pltpu.make_async_copy(w_hbm.at[wid, 0], wv.at[pl.ds(0, BCH)], bsem).wait()

        def gather_start(row, p):
            pltpu.async_copy(t_hbm.at[rv.at[row]], bufs[p], gsems[p])

        def gather_wait(p):
            pltpu.make_async_copy(t_hbm.at[rv.at[0]], bufs[p], gsems[p]).wait()

        def scatter_start(row, p):
            pltpu.async_copy(bufs[p], accum.at[cv.at[row]], ssems[p], add=True)

        def scatter_wait(p):
            pltpu.make_async_copy(bufs[p], accum.at[cv.at[0]], ssems[p]).wait()

        def process(j, p):
            row = lax.rem(j, 2 * BCH)
            nxt = j + 1
            b = lax.div(j, BCH)

            @pl.when(jnp.logical_and(lax.rem(j, BCH) == 2, b < NBLK - 1))
            def _():
                load_block(b + 1, lax.rem(b + 1, 2) * BCH)

            @pl.when(j > 0)
            def _():
                scatter_wait(1 - p)

            @pl.when(jnp.logical_and(lax.rem(nxt, BCH) == 0, nxt < NCHUNK))
            def _():
                wait_block()

            @pl.when(nxt < NCHUNK)
            def _():
                gather_start(lax.rem(nxt, 2 * BCH), 1 - p)

            for t in range(CH // 16):
                r16 = rv[row, pl.ds(t * 16, 16)]
                s16 = wv[row, pl.ds(t * 16, 16)] * plsc.load_gather(gv, [r16])
                sv[pl.ds(t * 16, 16)] = s16

            gather_wait(p)
            buf = bufs[p]

            @pl.loop(0, CH, unroll=4)
            def _(i):
                sc = plsc.load_gather(sv, [jnp.broadcast_to(i, (16,))])
                for k in range(D // 16):
                    sl = pl.ds(k * 16, 16)
                    buf[i, sl] = buf[i, sl] * sc

            scatter_start(row, p)

        load_block(0, 0)
        wait_block()
        gather_start(0, 0)

        @pl.loop(0, NCHUNK // 2)
        def _(tt):
            j = 2 * tt
            process(j, 0)
            process(j + 1, 1)

        process(jnp.int32(NCHUNK - 1), 0)
        scatter_wait(0)

        plsc.subcore_barrier()
        pltpu.sync_copy(
            accum.at[pl.ds(sid * RPT, RPT)],
            out_hbm.at[cid, pl.ds(sid * RPT, RPT)],
        )

    return agg_k(table, rows3, cols3, ew3, g)


def _dot(a, b):
    return lax.dot_general(
        a, b, (((1,), (0,)), ((), ())),
        precision=lax.Precision.HIGHEST,
        preferred_element_type=jnp.float32,
    )


def _mm(x, W):
    def body(x_ref, w_ref, o_ref):
        o_ref[...] = _dot(x_ref[...], w_ref[...])

    return pl.pallas_call(
        body,
        grid=(N // BM,),
        in_specs=[
            pl.BlockSpec((BM, D), lambda i: (i, 0)),
            pl.BlockSpec((D, D), lambda i: (0, 0)),
        ],
        out_specs=pl.BlockSpec((BM, D), lambda i: (i, 0)),
        out_shape=jax.ShapeDtypeStruct((N, D), jnp.float32),
    )(x, W)


def _mid(A, H1, g2, b1, W2):
    def body(a_ref, h_ref, g_ref, b_ref, w_ref, o_ref):
        g = g_ref[...]
        o = g * (a_ref[0] + a_ref[1] + g * h_ref[...]) + b_ref[...]
        o = jnp.where(o >= 0, o, 0.01 * o)
        o_ref[...] = _dot(o, w_ref[...])

    return pl.pallas_call(
        body,
        grid=(N // BM,),
        in_specs=[
            pl.BlockSpec((NC, BM, D), lambda i: (0, i, 0)),
            pl.BlockSpec((BM, D), lambda i: (i, 0)),
            pl.BlockSpec((BM, 1), lambda i: (i, 0)),
            pl.BlockSpec((1, D), lambda i: (0, 0)),
            pl.BlockSpec((D, D), lambda i: (0, 0)),
        ],
        out_specs=pl.BlockSpec((BM, D), lambda i: (i, 0)),
        out_shape=jax.ShapeDtypeStruct((N, D), jnp.float32),
    )(A, H1, g2, b1, W2)


def _final(A, H2, g2, b2):
    def body(a_ref, h_ref, g_ref, b_ref, o_ref):
        g = g_ref[...]
        o_ref[...] = g * (a_ref[0] + a_ref[1] + g * h_ref[...]) + b_ref[...]

    return pl.pallas_call(
        body,
        grid=(N // BM,),
        in_specs=[
            pl.BlockSpec((NC, BM, D), lambda i: (0, i, 0)),
            pl.BlockSpec((BM, D), lambda i: (i, 0)),
            pl.BlockSpec((BM, 1), lambda i: (i, 0)),
            pl.BlockSpec((1, D), lambda i: (0, 0)),
        ],
        out_specs=pl.BlockSpec((BM, D), lambda i: (i, 0)),
        out_shape=jax.ShapeDtypeStruct((N, D), jnp.float32),
    )(A, H2, g2, b2)


def kernel(x, edge_index, edge_weight, W1, b1, W2, b2):
    rows = edge_index[0].astype(jnp.int32)
    cols = edge_index[1].astype(jnp.int32)
    ew = edge_weight.astype(jnp.float32)

    rows3 = rows.reshape(NW, NBLK, BCH, CH)
    cols3 = cols.reshape(NW, NBLK, BCH, CH)
    ew3 = ew.reshape(NW, NBLK, BCH, CH)
    cols2 = cols.reshape(NS, DEGC, 16)
    ew2 = ew.reshape(NS, DEGC, 16)

    g_full = _deg_g(cols2, ew2)
    H1 = _mm(x, W1)
    A1 = _agg(H1, rows3, cols3, ew3, g_full)

    g2 = g_full[:N].reshape(N, 1)
    H2 = _mid(A1, H1, g2, b1.reshape(1, D), W2)
    A2 = _agg(H2, rows3, cols3, ew3, g_full)
    return _final(A2, H2, g2, b2.reshape(1, D))


# speedup vs baseline: 27.2028x; 1.1476x over previous
import dataclasses
import functools

import jax
import jax.numpy as jnp
from jax import lax
from jax.experimental import pallas as pl
from jax.experimental.pallas import tpu as pltpu
from jax.experimental.pallas import tpu_sc as plsc

N = 10000
E = 320000
D = 128
NP = 10240
NC, NS = 2, 16
NW = NC * NS
EPW = E // NW
CH = 80
NCHUNK = EPW // CH
BCH = 25
NBLK = NCHUNK // BCH
EPT = E // NS
DEGC = EPT // 16
SLICE = NP // NS
RPT = N // NS
ZROWS = 25
BM = 1000


def _sc_mesh():
    return plsc.VectorSubcoreMesh(
        core_axis_name="c", subcore_axis_name="s", num_cores=NC, num_subcores=NS
    )


def _sc_params():
    cp = pltpu.CompilerParams()
    fields = pltpu.CompilerParams.__dataclass_fields__
    if "needs_layout_passes" in fields:
        cp = dataclasses.replace(cp, needs_layout_passes=False)
    if "use_tc_tiling_on_sc" in fields:
        cp = dataclasses.replace(cp, use_tc_tiling_on_sc=False)
    return cp


def _rsqrt16(x):
    i = plsc.bitcast(x, jnp.int32)
    y = plsc.bitcast(jnp.int32(0x5F3759DF) - (i >> 1), jnp.float32)
    for _ in range(4):
        y = y * (1.5 - 0.5 * x * y * y)
    return y


def _deg_g(cols2, ew2):
    @functools.partial(
        pl.kernel,
        out_type=jax.ShapeDtypeStruct((NP,), jnp.float32),
        mesh=_sc_mesh(),
        scratch_types=[
            pltpu.VMEM((DEGC, 16), jnp.int32),
            pltpu.VMEM((DEGC, 16), jnp.float32),
            pltpu.VMEM((NP,), jnp.float32),
            pltpu.VMEM((NS, SLICE), jnp.float32),
            pltpu.VMEM((SLICE,), jnp.float32),
            pltpu.VMEM_SHARED((NS, NP), jnp.float32),
            pltpu.SemaphoreType.DMA,
        ],
        compiler_params=_sc_params(),
    )
    def deg_k(cols_hbm, ew_hbm, g_hbm, colv, ewv, degl, redv, gv, degsh, sem):
        cid = lax.axis_index("c")
        sid = lax.axis_index("s")

        @pl.when(cid == 0)
        def _():
            pltpu.async_copy(cols_hbm.at[sid], colv, sem).wait()
            pltpu.async_copy(ew_hbm.at[sid], ewv, sem).wait()

            @pl.loop(0, NP // 16)
            def _(i):
                degl[pl.ds(i * 16, 16)] = jnp.zeros((16,), jnp.float32)

            @pl.loop(0, DEGC)
            def _(j):
                plsc.addupdate_scatter(degl, [colv[j]], ewv[j])

            pltpu.sync_copy(degl, degsh.at[sid])
            plsc.subcore_barrier()

            pltpu.sync_copy(degsh.at[:, pl.ds(sid * SLICE, SLICE)], redv)

            @pl.loop(0, SLICE // 16)
            def _(m):
                acc = jnp.full((16,), 1.0, jnp.float32)
                for p in range(NS):
                    acc = acc + redv[p, pl.ds(m * 16, 16)]
                gv[pl.ds(m * 16, 16)] = _rsqrt16(acc)

            pltpu.sync_copy(gv, g_hbm.at[pl.ds(sid * SLICE, SLICE)])

    return deg_k(cols2, ew2)


def _agg(table, rows3, cols3, ew3):
    NB = 3

    @functools.partial(
        pl.kernel,
        out_type=jax.ShapeDtypeStruct((NC, N, D), jnp.float32),
        mesh=_sc_mesh(),
        scratch_types=[
            pltpu.VMEM((2 * BCH, CH), jnp.int32),
            pltpu.VMEM((2 * BCH, CH), jnp.int32),
            pltpu.VMEM((2 * BCH, CH), jnp.float32),
            pltpu.VMEM((CH, D), jnp.float32),
            pltpu.VMEM((CH, D), jnp.float32),
            pltpu.VMEM((CH, D), jnp.float32),
            pltpu.VMEM((ZROWS, D), jnp.float32),
            pltpu.VMEM_SHARED((N, D), jnp.float32),
            pltpu.SemaphoreType.DMA,
            pltpu.SemaphoreType.DMA,
            pltpu.SemaphoreType.DMA,
            pltpu.SemaphoreType.DMA,
            pltpu.SemaphoreType.DMA,
            pltpu.SemaphoreType.DMA,
            pltpu.SemaphoreType.DMA,
        ],
        compiler_params=_sc_params(),
    )
    def agg_k(t_hbm, r_hbm, c_hbm, w_hbm, out_hbm,
              rv, cv, wv, buf0, buf1, buf2, zbuf, accum,
              bsem, gsem0, gsem1, gsem2, ssem0, ssem1, ssem2):
        cid = lax.axis_index("c")
        sid = lax.axis_index("s")
        wid = cid * NS + sid
        bufs = (buf0, buf1, buf2)
        gsems = (gsem0, gsem1, gsem2)
        ssems = (ssem0, ssem1, ssem2)

        @pl.loop(0, ZROWS)
        def _(i):
            for k in range(D // 16):
                zbuf[i, pl.ds(k * 16, 16)] = jnp.zeros((16,), jnp.float32)

        @pl.loop(0, RPT // ZROWS)
        def _(m):
            pltpu.async_copy(
                zbuf, accum.at[pl.ds(sid * RPT + m * ZROWS, ZROWS)], bsem)

        @pl.loop(0, RPT // ZROWS)
        def _(m):
            pltpu.make_async_copy(
                zbuf, accum.at[pl.ds(0, ZROWS)], bsem).wait()
        plsc.subcore_barrier()

        def load_block(b, off):
            pltpu.async_copy(r_hbm.at[wid, b], rv.at[pl.ds(off, BCH)], bsem)
            pltpu.async_copy(c_hbm.at[wid, b], cv.at[pl.ds(off, BCH)], bsem)
            pltpu.async_copy(w_hbm.at[wid, b], wv.at[pl.ds(off, BCH)], bsem)

        def wait_block():
            pltpu.make_async_copy(r_hbm.at[wid, 0], rv.at[pl.ds(0, BCH)], bsem).wait()
            pltpu.make_async_copy(c_hbm.at[wid, 0], cv.at[pl.ds(0, BCH)], bsem).wait()
            pltpu.make_async_copy(w_hbm.at[wid, 0], wv.at[pl.ds(0, BCH)], bsem).wait()

        def gather_start(row, p):
            pltpu.async_copy(t_hbm.at[rv.at[row]], bufs[p], gsems[p])

        def gather_wait(p):
            pltpu.make_async_copy(t_hbm.at[rv.at[0]], bufs[p], gsems[p]).wait()

        def scatter_start(row, p):
            pltpu.async_copy(bufs[p], accum.at[cv.at[row]], ssems[p], add=True)

        def scatter_wait(p):
            pltpu.make_async_copy(bufs[p], accum.at[cv.at[0]], ssems[p]).wait()

        def process(j, p):
            row = lax.rem(j, 2 * BCH)
            nxt = j + 1
            b = lax.div(j, BCH)
            pn = (p + 1) % NB

            @pl.when(jnp.logical_and(lax.rem(j, BCH) == 3, b < NBLK - 1))
            def _():
                load_block(b + 1, lax.rem(b + 1, 2) * BCH)

            @pl.when(j >= 2)
            def _():
                scatter_wait(pn)

            @pl.when(jnp.logical_and(lax.rem(nxt, BCH) == 0, nxt < NCHUNK))
            def _():
                wait_block()

            @pl.when(nxt < NCHUNK)
            def _():
                gather_start(lax.rem(nxt, 2 * BCH), pn)

            gather_wait(p)
            buf = bufs[p]

            @pl.loop(0, CH, unroll=4)
            def _(i):
                sc = plsc.load_gather(
                    wv, [jnp.broadcast_to(row, (16,)), jnp.broadcast_to(i, (16,))])
                for k in range(D // 16):
                    sl = pl.ds(k * 16, 16)
                    buf[i, sl] = buf[i, sl] * sc

            scatter_start(row, p)

        load_block(0, 0)
        wait_block()
        gather_start(0, 0)

        @pl.loop(0, NCHUNK // NB)
        def _(tt):
            j = NB * tt
            process(j, 0)
            process(j + 1, 1)
            process(j + 2, 2)

        process(jnp.int32(NCHUNK - 2), 0)
        process(jnp.int32(NCHUNK - 1), 1)
        scatter_wait(0)
        scatter_wait(1)

        plsc.subcore_barrier()
        pltpu.sync_copy(
            accum.at[pl.ds(sid * RPT, RPT)],
            out_hbm.at[cid, pl.ds(sid * RPT, RPT)],
        )

    return agg_k(table, rows3, cols3, ew3)


def _dot(a, b):
    return lax.dot_general(
        a, b, (((1,), (0,)), ((), ())),
        precision=lax.Precision.HIGHEST,
        preferred_element_type=jnp.float32,
    )


def _mm(x, W):
    def body(x_ref, w_ref, o_ref):
        o_ref[...] = _dot(x_ref[...], w_ref[...])

    return pl.pallas_call(
        body,
        grid=(N // BM,),
        in_specs=[
            pl.BlockSpec((BM, D), lambda i: (i, 0)),
            pl.BlockSpec((D, D), lambda i: (0, 0)),
        ],
        out_specs=pl.BlockSpec((BM, D), lambda i: (i, 0)),
        out_shape=jax.ShapeDtypeStruct((N, D), jnp.float32),
    )(x, W)


def _scale_t(H, g2):
    def body(h_ref, g_ref, o_ref):
        o_ref[...] = g_ref[...] * h_ref[...]

    return pl.pallas_call(
        body,
        grid=(N // BM,),
        in_specs=[
            pl.BlockSpec((BM, D), lambda i: (i, 0)),
            pl.BlockSpec((BM, 1), lambda i: (i, 0)),
        ],
        out_specs=pl.BlockSpec((BM, D), lambda i: (i, 0)),
        out_shape=jax.ShapeDtypeStruct((N, D), jnp.float32),
    )(H, g2)


def _mid(A, T1, g2, b1, W2):
    def body(a_ref, h_ref, g_ref, b_ref, w_ref, o_ref):
        g = g_ref[...]
        o = g * (a_ref[0] + a_ref[1] + h_ref[...]) + b_ref[...]
        o = jnp.where(o >= 0, o, 0.01 * o)
        o_ref[...] = g * _dot(o, w_ref[...])

    return pl.pallas_call(
        body,
        grid=(N // BM,),
        in_specs=[
            pl.BlockSpec((NC, BM, D), lambda i: (0, i, 0)),
            pl.BlockSpec((BM, D), lambda i: (i, 0)),
            pl.BlockSpec((BM, 1), lambda i: (i, 0)),
            pl.BlockSpec((1, D), lambda i: (0, 0)),
            pl.BlockSpec((D, D), lambda i: (0, 0)),
        ],
        out_specs=pl.BlockSpec((BM, D), lambda i: (i, 0)),
        out_shape=jax.ShapeDtypeStruct((N, D), jnp.float32),
    )(A, T1, g2, b1, W2)


def _final(A, T2, g2, b2):
    def body(a_ref, h_ref, g_ref, b_ref, o_ref):
        g = g_ref[...]
        o_ref[...] = g * (a_ref[0] + a_ref[1] + h_ref[...]) + b_ref[...]

    return pl.pallas_call(
        body,
        grid=(N // BM,),
        in_specs=[
            pl.BlockSpec((NC, BM, D), lambda i: (0, i, 0)),
            pl.BlockSpec((BM, D), lambda i: (i, 0)),
            pl.BlockSpec((BM, 1), lambda i: (i, 0)),
            pl.BlockSpec((1, D), lambda i: (0, 0)),
        ],
        out_specs=pl.BlockSpec((BM, D), lambda i: (i, 0)),
        out_shape=jax.ShapeDtypeStruct((N, D), jnp.float32),
    )(A, T2, g2, b2)


def kernel(x, edge_index, edge_weight, W1, b1, W2, b2):
    rows = edge_index[0].astype(jnp.int32)
    cols = edge_index[1].astype(jnp.int32)
    ew = edge_weight.astype(jnp.float32)

    rows3 = rows.reshape(NW, NBLK, BCH, CH)
    cols3 = cols.reshape(NW, NBLK, BCH, CH)
    ew3 = ew.reshape(NW, NBLK, BCH, CH)
    cols2 = cols.reshape(NS, DEGC, 16)
    ew2 = ew.reshape(NS, DEGC, 16)

    g_full = _deg_g(cols2, ew2)
    H1 = _mm(x, W1)
    g2 = g_full[:N].reshape(N, 1)
    T1 = _scale_t(H1, g2)
    A1 = _agg(T1, rows3, cols3, ew3)

    T2 = _mid(A1, T1, g2, b1.reshape(1, D), W2)
    A2 = _agg(T2, rows3, cols3, ew3)
    return _final(A2, T2, g2, b2.reshape(1, D))
